# baseline (speedup 1.0000x reference)
import jax
import jax.numpy as jnp
from jax import lax
from jax.experimental import pallas as pl
from jax.experimental.pallas import tpu as pltpu
from jax.experimental.pallas import tpu_sc as plsc

NUM_FIELDS = 26
VOCAB_PER_FIELD = 100000
NUM_CLASSES = 8
BATCH = 16384

L = 16
NW = 32
WPS = 16
ROWS_PER_OP = 128
SUBCH_PER_W = BATCH // (NW * ROWS_PER_OP)
OPS_PER_W = SUBCH_PER_W * NUM_FIELDS
ROWS_PER_W = SUBCH_PER_W * ROWS_PER_OP


def _sc_body(idx_hbm, table_hbm, out_hbm, idx_v, rows_v, ident_v, tmp_v,
             acc_sh, sem):
    cid = lax.axis_index("c")
    sid = lax.axis_index("s")
    wid = sid * 2 + cid
    base_sub = wid * SUBCH_PER_W

    pltpu.sync_copy(idx_hbm.at[pl.ds(base_sub, SUBCH_PER_W)], idx_v)

    lanes = lax.iota(jnp.int32, L)

    for j in range(SUBCH_PER_W):
        for v in range(ROWS_PER_OP // L):
            ident_v[j, pl.ds(v * L, L)] = lanes + (
                sid * ROWS_PER_W + j * ROWS_PER_OP + v * L)

    def _off_body(k, _):
        j = k // NUM_FIELDS
        f = k % NUM_FIELDS
        off = jnp.full((L,), f * VOCAB_PER_FIELD, jnp.int32)
        for v in range(ROWS_PER_OP // L):
            sl = pl.ds(v * L, L)
            idx_v[j, f, sl] = idx_v[j, f, sl] + off
        return 0

    lax.fori_loop(0, OPS_PER_W, _off_body, 0)

    def _gather_body(k, _):
        j = k // NUM_FIELDS
        f = k % NUM_FIELDS
        cp = pltpu.make_async_copy(table_hbm.at[idx_v.at[j, f]], rows_v, sem)
        cp.start()
        cp.wait()

        @pl.when(f == 0)
        def _init():
            pltpu.sync_copy(rows_v, acc_sh.at[ident_v.at[j]])

        @pl.when(f != 0)
        def _accum():
            pltpu.sync_copy(rows_v, acc_sh.at[ident_v.at[j]], add=True)

        return 0

    lax.fori_loop(0, OPS_PER_W, _gather_body, 0)

    pltpu.sync_copy(acc_sh.at[pl.ds(sid * ROWS_PER_W, ROWS_PER_W)], tmp_v)
    pltpu.sync_copy(tmp_v, out_hbm.at[pl.ds(wid * ROWS_PER_W, ROWS_PER_W)])


@jax.jit
def _run(idx_r, table):
    mesh = plsc.VectorSubcoreMesh(core_axis_name="c", subcore_axis_name="s")
    call = pl.kernel(
        _sc_body,
        mesh=mesh,
        compiler_params=pltpu.CompilerParams(use_tc_tiling_on_sc=False),
        out_type=jax.ShapeDtypeStruct((BATCH, NUM_CLASSES), jnp.float32),
        scratch_types=[
            pltpu.VMEM((SUBCH_PER_W, NUM_FIELDS, ROWS_PER_OP), jnp.int32),
            pltpu.VMEM((ROWS_PER_OP, NUM_CLASSES), jnp.float32),
            pltpu.VMEM((SUBCH_PER_W, ROWS_PER_OP), jnp.int32),
            pltpu.VMEM((ROWS_PER_W, NUM_CLASSES), jnp.float32),
            pltpu.VMEM_SHARED((WPS * ROWS_PER_W, NUM_CLASSES), jnp.float32),
            pltpu.SemaphoreType.DMA,
        ],
    )
    return call(idx_r, table)


def kernel(indices, table):
    idx_r = indices.astype(jnp.int32).reshape(
        BATCH // ROWS_PER_OP, ROWS_PER_OP, NUM_FIELDS).transpose(0, 2, 1)
    return _run(idx_r, table)


# speedup vs baseline: 1.0336x; 1.0336x over previous
import jax
import jax.numpy as jnp
from jax import lax
from jax.experimental import pallas as pl
from jax.experimental.pallas import tpu as pltpu
from jax.experimental.pallas import tpu_sc as plsc

NUM_FIELDS = 26
VOCAB_PER_FIELD = 100000
NUM_CLASSES = 8
BATCH = 16384

L = 16
NW = 32
ROWS_PER_OP = 128
SUBCH_PER_W = BATCH // (NW * ROWS_PER_OP)
OPS_PER_W = SUBCH_PER_W * NUM_FIELDS
ROWS_PER_W = SUBCH_PER_W * ROWS_PER_OP


def _sc_body(idx_hbm, table_hbm, out_hbm, idx_v, rows_v, ident_v, tmp_v,
             acc_sh, sem0, sem1):
    cid = lax.axis_index("c")
    sid = lax.axis_index("s")
    wid = sid * 2 + cid
    base_sub = wid * SUBCH_PER_W

    pltpu.sync_copy(idx_hbm.at[pl.ds(base_sub, SUBCH_PER_W)], idx_v)

    lanes = lax.iota(jnp.int32, L)

    for j in range(SUBCH_PER_W):
        for v in range(ROWS_PER_OP // L):
            ident_v[j, pl.ds(v * L, L)] = lanes + (
                sid * ROWS_PER_W + j * ROWS_PER_OP + v * L)

    def _off_body(k, _):
        j = k // NUM_FIELDS
        f = k % NUM_FIELDS
        off = jnp.full((L,), f * VOCAB_PER_FIELD, jnp.int32)
        for v in range(ROWS_PER_OP // L):
            sl = pl.ds(v * L, L)
            idx_v[j, f, sl] = idx_v[j, f, sl] + off
        return 0

    lax.fori_loop(0, OPS_PER_W, _off_body, 0)

    def _start_gather(k, slot_ref, sem):
        j = k // NUM_FIELDS
        f = k % NUM_FIELDS
        pltpu.make_async_copy(
            table_hbm.at[idx_v.at[j, f]], slot_ref, sem).start()

    def _accumulate(k, slot_ref, sem):
        j = k // NUM_FIELDS
        f = k % NUM_FIELDS
        pltpu.make_async_copy(
            table_hbm.at[idx_v.at[j, f]], slot_ref, sem).wait()

        @pl.when(f == 0)
        def _init():
            pltpu.sync_copy(slot_ref, acc_sh.at[ident_v.at[j]])

        @pl.when(f != 0)
        def _accum():
            pltpu.sync_copy(slot_ref, acc_sh.at[ident_v.at[j]], add=True)

    _start_gather(0, rows_v.at[0], sem0)

    def _pipe_body(kk, _):
        k0 = 2 * kk
        _start_gather(k0 + 1, rows_v.at[1], sem1)
        _accumulate(k0, rows_v.at[0], sem0)

        @pl.when(kk < OPS_PER_W // 2 - 1)
        def _next():
            _start_gather(k0 + 2, rows_v.at[0], sem0)

        _accumulate(k0 + 1, rows_v.at[1], sem1)
        return 0

    lax.fori_loop(0, OPS_PER_W // 2, _pipe_body, 0)

    pltpu.sync_copy(acc_sh.at[pl.ds(sid * ROWS_PER_W, ROWS_PER_W)], tmp_v)
    pltpu.sync_copy(tmp_v, out_hbm.at[pl.ds(wid * ROWS_PER_W, ROWS_PER_W)])


@jax.jit
def _run(idx_r, table):
    mesh = plsc.VectorSubcoreMesh(core_axis_name="c", subcore_axis_name="s")
    call = pl.kernel(
        _sc_body,
        mesh=mesh,
        compiler_params=pltpu.CompilerParams(use_tc_tiling_on_sc=False),
        out_type=jax.ShapeDtypeStruct((BATCH, NUM_CLASSES), jnp.float32),
        scratch_types=[
            pltpu.VMEM((SUBCH_PER_W, NUM_FIELDS, ROWS_PER_OP), jnp.int32),
            pltpu.VMEM((2, ROWS_PER_OP, NUM_CLASSES), jnp.float32),
            pltpu.VMEM((SUBCH_PER_W, ROWS_PER_OP), jnp.int32),
            pltpu.VMEM((ROWS_PER_W, NUM_CLASSES), jnp.float32),
            pltpu.VMEM_SHARED((16 * ROWS_PER_W, NUM_CLASSES), jnp.float32),
            pltpu.SemaphoreType.DMA,
            pltpu.SemaphoreType.DMA,
        ],
    )
    return call(idx_r, table)


def kernel(indices, table):
    idx_r = indices.astype(jnp.int32).reshape(
        BATCH // ROWS_PER_OP, ROWS_PER_OP, NUM_FIELDS).transpose(0, 2, 1)
    return _run(idx_r, table)
